# baseline (speedup 1.0000x reference)
import jax
import jax.numpy as jnp
from jax.experimental import pallas as pl
from jax.experimental.pallas import tpu as pltpu

_N_CLASSES = 1000
_B0, _B1 = 1024, 20
_ROWS = 128
_NQ = 4
_RQ = _ROWS // _NQ
_NSTEPS = _B0 // _ROWS


def _copies(scratch, out_ref, sem, g, b):
    return [
        pltpu.make_async_copy(
            scratch.at[b, pl.ds(q * _RQ, _RQ)],
            out_ref.at[pl.ds(g * _ROWS + q * _RQ, _RQ)],
            sem.at[b, q],
        )
        for q in range(_NQ)
    ]


def _onehot_body(idx_ref, out_ref, scratch, sem):
    g = pl.program_id(0)
    b = g % 2

    @pl.when(g >= 2)
    def _wait_prev_same_buf():
        for c in _copies(scratch, out_ref, sem, g - 2, b):
            c.wait()

    iota = jax.lax.broadcasted_iota(jnp.int32, (_ROWS, _B1, _N_CLASSES), 2)
    scratch[b] = (iota == idx_ref[...][:, :, None]).astype(jnp.float32)

    for c in _copies(scratch, out_ref, sem, g, b):
        c.start()

    @pl.when(g == _NSTEPS - 1)
    def _drain():
        for c in _copies(scratch, out_ref, sem, g - 1, 1 - b):
            c.wait()
        for c in _copies(scratch, out_ref, sem, g, b):
            c.wait()


def kernel(input):
    idx = input.astype(jnp.int32)
    return pl.pallas_call(
        _onehot_body,
        grid=(_NSTEPS,),
        in_specs=[pl.BlockSpec((_ROWS, _B1), lambda i: (i, 0))],
        out_specs=pl.BlockSpec(memory_space=pl.ANY),
        out_shape=jax.ShapeDtypeStruct((_B0, _B1, _N_CLASSES), jnp.float32),
        scratch_shapes=[
            pltpu.VMEM((2, _ROWS, _B1, _N_CLASSES), jnp.float32),
            pltpu.SemaphoreType.DMA((2, _NQ)),
        ],
        compiler_params=pltpu.CompilerParams(
            dimension_semantics=("arbitrary",),
        ),
    )(idx)


# speedup vs baseline: 3.3934x; 3.3934x over previous
import jax
import jax.numpy as jnp
from jax.experimental import pallas as pl
from jax.experimental.pallas import tpu as pltpu

_N_CLASSES = 1024
_B0, _B1 = 1024, 24
_ROWS = 128


def _onehot_body(idx_ref, out_ref):
    iota = jax.lax.broadcasted_iota(jnp.int32, out_ref.shape, 2)
    out_ref[...] = (iota == idx_ref[...][:, :, None]).astype(jnp.float32)


def kernel(input):
    idx = input.astype(jnp.int32)
    idx = jnp.pad(idx, ((0, 0), (0, 4)))
    return pl.pallas_call(
        _onehot_body,
        grid=(_B0 // _ROWS,),
        in_specs=[pl.BlockSpec((_ROWS, _B1), lambda i: (i, 0))],
        out_specs=pl.BlockSpec((_ROWS, _B1, _N_CLASSES), lambda i: (i, 0, 0)),
        out_shape=jax.ShapeDtypeStruct((_B0, _B1, _N_CLASSES), jnp.float32),
    )(idx)


# speedup vs baseline: 4.3522x; 1.2825x over previous
import jax
import jax.numpy as jnp
from jax.experimental import pallas as pl

_N_CLASSES = 1000
_B0, _B1 = 1024, 20


def _onehot_body(idx_ref, out_ref):
    iota = jax.lax.broadcasted_iota(jnp.int32, out_ref.shape, 1)
    out_ref[...] = (iota == idx_ref[...]).astype(jnp.float32)


def kernel(input):
    idx_t = jnp.transpose(input.astype(jnp.int32))
    idx3 = jnp.reshape(idx_t, (_B1, 1, _B0))
    out_t = pl.pallas_call(
        _onehot_body,
        grid=(_B1,),
        in_specs=[pl.BlockSpec((1, 1, _B0), lambda j: (j, 0, 0))],
        out_specs=pl.BlockSpec((1, _N_CLASSES, _B0), lambda j: (j, 0, 0)),
        out_shape=jax.ShapeDtypeStruct((_B1, _N_CLASSES, _B0), jnp.float32),
    )(idx3)
    return jnp.transpose(out_t, (2, 0, 1))


# speedup vs baseline: 4.5761x; 1.0515x over previous
import jax
import jax.numpy as jnp
from jax.experimental import pallas as pl

_N_CLASSES = 1000
_B0, _B1 = 1024, 20


def _onehot_body(idx_ref, out_ref):
    j = pl.program_id(0)
    row = idx_ref[pl.ds(j, 1), :]
    iota = jax.lax.broadcasted_iota(jnp.int32, out_ref.shape, 1)
    out_ref[...] = (iota == row[:, None, :]).astype(jnp.float32)


def kernel(input):
    idx_t = jnp.transpose(input.astype(jnp.int32))
    out_t = pl.pallas_call(
        _onehot_body,
        grid=(_B1,),
        in_specs=[pl.BlockSpec((_B1, _B0), lambda j: (0, 0))],
        out_specs=pl.BlockSpec((1, _N_CLASSES, _B0), lambda j: (j, 0, 0)),
        out_shape=jax.ShapeDtypeStruct((_B1, _N_CLASSES, _B0), jnp.float32),
    )(idx_t)
    return jnp.transpose(out_t, (2, 0, 1))
